# baseline (speedup 1.0000x reference)
import functools

import jax
import jax.numpy as jnp
from jax import lax
from jax.experimental import pallas as pl
from jax.experimental.pallas import tpu as pltpu
from jax.experimental.pallas import tpu_sc as plsc

_NC = 2
_NS = 16
_NW = _NC * _NS

_DEG = 16
_H = 32
_DF = 32
_PK = 4
_LW = _PK * _H

_CH = 128
_KG = 8


def _phase0_body(y_ref, w1a_ref, w1b4_ref, b14_ref, a_ref, b_ref):
    yb = y_ref[...]
    a_ref[...] = jnp.dot(yb, w1a_ref[...], preferred_element_type=jnp.float32)
    b_ref[...] = (
        jnp.dot(yb, w1b4_ref[...], preferred_element_type=jnp.float32) + b14_ref[...]
    )


def _phase2_body(ga_ref, gf_ref, b4_ref, w2d_ref, b2_ref, out_ref):
    nb = b4_ref.shape[0]
    b4 = b4_ref[...]
    w2d = w2d_ref[...]
    acc = jnp.zeros((nb, _LW), jnp.float32)
    for c in range(_DEG // _PK):
        a4 = ga_ref[pl.ds(c * nb, nb), :]
        f4 = gf_ref[pl.ds(c * nb, nb), :]
        h = jax.nn.gelu(a4 + b4)
        k = jnp.dot(h, w2d, preferred_element_type=jnp.float32) + b2_ref[...]
        acc = acc + k * f4
    s = (
        acc[:, 0:_H] + acc[:, _H : 2 * _H] + acc[:, 2 * _H : 3 * _H] + acc[:, 3 * _H :]
    )
    out_ref[...] = s * (1.0 / _DEG)


def _sc_gather2(ta, tf, idx_pad, n_edges_pad):
    rows_per_tile = n_edges_pad // _NW
    rows_per_step = _KG * _CH
    steps = rows_per_tile // rows_per_step

    mesh = plsc.VectorSubcoreMesh(core_axis_name="c", subcore_axis_name="s")

    prows_per_step = rows_per_step // _PK

    @functools.partial(
        pl.kernel,
        out_type=(
            jax.ShapeDtypeStruct((n_edges_pad, _H), jnp.float32),
            jax.ShapeDtypeStruct((n_edges_pad, _DF), jnp.float32),
        ),
        mesh=mesh,
        scratch_types=[
            pltpu.VMEM((rows_per_step,), jnp.int32),
            pltpu.VMEM((rows_per_step, _H), jnp.float32),
            pltpu.VMEM((rows_per_step, _DF), jnp.float32),
            pltpu.SemaphoreType.DMA,
        ],
        compiler_params=pltpu.CompilerParams(use_tc_tiling_on_sc=False),
    )
    def gather_kernel(ta_hbm, tf_hbm, idx_hbm, ga_hbm, gf_hbm, idx_v, ra_v, rf_v, sem):
        wid = lax.axis_index("s") * _NC + lax.axis_index("c")
        row_base = wid * rows_per_tile

        def step(t, carry):
            off = row_base + t * rows_per_step
            poff = off // _PK
            pltpu.sync_copy(idx_hbm.at[pl.ds(off, rows_per_step)], idx_v)
            cpa = pltpu.async_copy(ta_hbm.at[idx_v], ra_v, sem)
            cpf = pltpu.async_copy(tf_hbm.at[idx_v], rf_v, sem)
            cpa.wait()
            cpf.wait()
            pltpu.sync_copy(ra_v, ga_hbm.at[pl.ds(off, rows_per_step)])
            pltpu.sync_copy(rf_v, gf_hbm.at[pl.ds(off, rows_per_step)])
            return carry

        lax.fori_loop(0, steps, step, None)

    return gather_kernel(ta, tf, idx_pad)


def kernel(y, f_y, neighbors_index, neighbors_row_splits, W1, b1, W2, b2):
    del neighbors_row_splits
    n = y.shape[0]
    e = neighbors_index.shape[0]
    idx = neighbors_index.astype(jnp.int32)
    nb = 1000
    idx_perm = (
        idx.reshape(n // nb, nb, _DEG // _PK, _PK)
        .transpose(0, 2, 1, 3)
        .reshape(-1)
    )
    step_rows = _NW * _KG * _CH
    e_pad = ((e + step_rows - 1) // step_rows) * step_rows
    idx_pad = jnp.pad(idx_perm, (0, e_pad - e))

    w1a = W1[:3]
    w1b4 = jnp.tile(W1[3:], (1, _PK))
    b14 = jnp.tile(b1, _PK).reshape(1, _LW)
    w2d = jnp.kron(jnp.eye(_PK, dtype=jnp.float32), W2)
    b2r = jnp.tile(b2, _PK).reshape(1, _LW)

    nb0 = 1000
    a_tab, b_tab = pl.pallas_call(
        _phase0_body,
        grid=(n // nb0,),
        in_specs=[
            pl.BlockSpec((nb0, 3), lambda i: (i, 0)),
            pl.BlockSpec((3, _H), lambda i: (0, 0)),
            pl.BlockSpec((3, _LW), lambda i: (0, 0)),
            pl.BlockSpec((1, _LW), lambda i: (0, 0)),
        ],
        out_specs=[
            pl.BlockSpec((nb0, _H), lambda i: (i, 0)),
            pl.BlockSpec((nb0, _LW), lambda i: (i, 0)),
        ],
        out_shape=[
            jax.ShapeDtypeStruct((n, _H), jnp.float32),
            jax.ShapeDtypeStruct((n, _LW), jnp.float32),
        ],
    )(y, w1a, w1b4, b14)

    ga, gf = _sc_gather2(a_tab, f_y, idx_pad, e_pad)
    ga4 = ga.reshape(e_pad // _PK, _LW)
    gf4 = gf.reshape(e_pad // _PK, _LW)

    nr = nb * _DEG // _PK
    out = pl.pallas_call(
        _phase2_body,
        grid=(n // nb,),
        in_specs=[
            pl.BlockSpec((nr, _LW), lambda i: (i, 0)),
            pl.BlockSpec((nr, _LW), lambda i: (i, 0)),
            pl.BlockSpec((nb, _LW), lambda i: (i, 0)),
            pl.BlockSpec((_LW, _LW), lambda i: (0, 0)),
            pl.BlockSpec((1, _LW), lambda i: (0, 0)),
        ],
        out_specs=pl.BlockSpec((nb, _DF), lambda i: (i, 0)),
        out_shape=jax.ShapeDtypeStruct((n, _DF), jnp.float32),
    )(ga4, gf4, b_tab, w2d, b2r)

    return out


# speedup vs baseline: 1.2423x; 1.2423x over previous
import functools

import jax
import jax.numpy as jnp
from jax import lax
from jax.experimental import pallas as pl
from jax.experimental.pallas import tpu as pltpu
from jax.experimental.pallas import tpu_sc as plsc

_NC = 2
_NS = 16
_NW = _NC * _NS

_DEG = 16
_H = 32
_DF = 32
_PK = 4
_LW = _PK * _H

_CH = 128
_KG = 8


def _phase0_body(y_ref, w1a_ref, w1b4_ref, b14_ref, a_ref, b_ref):
    yb = y_ref[...]
    a_ref[...] = jnp.dot(yb, w1a_ref[...], preferred_element_type=jnp.float32)
    b_ref[...] = (
        jnp.dot(yb, w1b4_ref[...], preferred_element_type=jnp.float32) + b14_ref[...]
    )


def _phase2_body(ga_ref, gf_ref, b4_ref, w2d_ref, b2_ref, out_ref):
    nb = b4_ref.shape[0]
    b4 = b4_ref[...]
    w2d = w2d_ref[...]
    acc = jnp.zeros((nb, _LW), jnp.float32)
    for c in range(_DEG // _PK):
        a4 = ga_ref[pl.ds(c * nb, nb), :]
        f4 = gf_ref[pl.ds(c * nb, nb), :]
        h = jax.nn.gelu(a4 + b4)
        k = jnp.dot(h, w2d, preferred_element_type=jnp.float32) + b2_ref[...]
        acc = acc + k * f4
    s = (
        acc[:, 0:_H] + acc[:, _H : 2 * _H] + acc[:, 2 * _H : 3 * _H] + acc[:, 3 * _H :]
    )
    out_ref[...] = s * (1.0 / _DEG)


def _sc_gather2(ta, tf, idx_t, n_nodes, nb):
    n_edges = n_nodes * _DEG
    prows = n_edges // _PK
    prows_per_tile = prows // _NW
    prows_per_step = 256
    steps = prows_per_tile // prows_per_step
    cg = _DEG // _PK

    mesh = plsc.VectorSubcoreMesh(core_axis_name="c", subcore_axis_name="s")

    @functools.partial(
        pl.kernel,
        out_type=(
            jax.ShapeDtypeStruct((prows, _LW), jnp.float32),
            jax.ShapeDtypeStruct((prows, _LW), jnp.float32),
        ),
        mesh=mesh,
        scratch_types=[
            [pltpu.VMEM((prows_per_step,), jnp.int32) for _ in range(_PK)],
            pltpu.VMEM((_PK * prows_per_step, _H), jnp.float32),
            pltpu.VMEM((_PK * prows_per_step, _DF), jnp.float32),
            pltpu.SemaphoreType.DMA,
        ],
        compiler_params=pltpu.CompilerParams(use_tc_tiling_on_sc=False),
    )
    def gather_kernel(ta_hbm, tf_hbm, idx_hbm, ga_hbm, gf_hbm, idx_vs, ra_v, rf_v, sem):
        wid = lax.axis_index("s") * _NC + lax.axis_index("c")
        prow_base = wid * prows_per_tile

        def step(t, carry):
            poff = prow_base + t * prows_per_step
            seg = poff // nb
            m0 = poff % nb
            c = seg % cg
            node0 = (seg // cg) * nb + m0
            for q in range(_PK):
                pltpu.sync_copy(
                    idx_hbm.at[c * _PK + q, pl.ds(node0, prows_per_step)],
                    idx_vs[q],
                )
            cps = []
            for q in range(_PK):
                sl = pl.ds(q * prows_per_step, prows_per_step)
                cps.append(pltpu.async_copy(ta_hbm.at[idx_vs[q]], ra_v.at[sl], sem))
                cps.append(pltpu.async_copy(tf_hbm.at[idx_vs[q]], rf_v.at[sl], sem))
            for cp in cps:
                cp.wait()
            for q in range(_PK):
                sl = pl.ds(q * prows_per_step, prows_per_step)
                dst = (pl.ds(poff, prows_per_step), pl.ds(q * _H, _H))
                pltpu.sync_copy(ra_v.at[sl], ga_hbm.at[dst])
                pltpu.sync_copy(rf_v.at[sl], gf_hbm.at[dst])
            return carry

        lax.fori_loop(0, steps, step, None)

    return gather_kernel(ta, tf, idx_t)


def kernel(y, f_y, neighbors_index, neighbors_row_splits, W1, b1, W2, b2):
    del neighbors_row_splits
    n = y.shape[0]
    e = neighbors_index.shape[0]
    idx = neighbors_index.astype(jnp.int32)
    nb = 1024
    n_pad = ((n + nb - 1) // nb) * nb
    y_p = jnp.pad(y, ((0, n_pad - n), (0, 0)))
    idx_t = jnp.pad(idx.reshape(n, _DEG), ((0, n_pad - n), (0, 0))).T

    w1a = W1[:3]
    w1b4 = jnp.tile(W1[3:], (1, _PK))
    b14 = jnp.tile(b1, _PK).reshape(1, _LW)
    w2d = jnp.kron(jnp.eye(_PK, dtype=jnp.float32), W2)
    b2r = jnp.tile(b2, _PK).reshape(1, _LW)

    nb0 = nb
    a_tab, b_tab = pl.pallas_call(
        _phase0_body,
        grid=(n_pad // nb0,),
        in_specs=[
            pl.BlockSpec((nb0, 3), lambda i: (i, 0)),
            pl.BlockSpec((3, _H), lambda i: (0, 0)),
            pl.BlockSpec((3, _LW), lambda i: (0, 0)),
            pl.BlockSpec((1, _LW), lambda i: (0, 0)),
        ],
        out_specs=[
            pl.BlockSpec((nb0, _H), lambda i: (i, 0)),
            pl.BlockSpec((nb0, _LW), lambda i: (i, 0)),
        ],
        out_shape=[
            jax.ShapeDtypeStruct((n_pad, _H), jnp.float32),
            jax.ShapeDtypeStruct((n_pad, _LW), jnp.float32),
        ],
    )(y_p, w1a, w1b4, b14)

    ga4, gf4 = _sc_gather2(a_tab, f_y, idx_t, n_pad, nb)

    nr = nb * _DEG // _PK
    out = pl.pallas_call(
        _phase2_body,
        grid=(n_pad // nb,),
        in_specs=[
            pl.BlockSpec((nr, _LW), lambda i: (i, 0)),
            pl.BlockSpec((nr, _LW), lambda i: (i, 0)),
            pl.BlockSpec((nb, _LW), lambda i: (i, 0)),
            pl.BlockSpec((_LW, _LW), lambda i: (0, 0)),
            pl.BlockSpec((1, _LW), lambda i: (0, 0)),
        ],
        out_specs=pl.BlockSpec((nb, _DF), lambda i: (i, 0)),
        out_shape=jax.ShapeDtypeStruct((n_pad, _DF), jnp.float32),
    )(ga4, gf4, b_tab, w2d, b2r)

    return out[:n]
